# baseline (speedup 1.0000x reference)
import jax
import jax.numpy as jnp
from jax import lax
from jax.experimental import pallas as pl
from jax.experimental.pallas import tpu as pltpu
from jax.experimental.pallas import tpu_sc as plsc

_B, _L, _D = 4096, 50, 64
_V = 1000000
_P = 475648
_TAIL0 = _P + 524288
_PROWS = 524288
_TBLK = 2048
_N = _B * _L
_NC, _NS = 2, 16
_NW = _NC * _NS
_PER_W = _N // _NW
_CH = 400
_NCH = _PER_W // _CH
_GS = 80
_NG = _CH // _GS
_SEQ_CH = _CH // _L
_EPS = 1e-5


def _gather16(v, idx):
    dnums = lax.GatherDimensionNumbers(
        offset_dims=(), collapsed_slice_dims=(0,), start_index_map=(0,))
    return lax.gather(v, idx.reshape(16, 1), dnums, slice_sizes=(1,),
                      mode=lax.GatherScatterMode.PROMISE_IN_BOUNDS)


_LANE = None


def _allreduce_sum(v, lane):
    for m in (1, 2, 4, 8):
        v = v + _gather16(v, lax.bitwise_xor(lane, jnp.int32(m)))
    return v


def _rsqrt(x):
    i = lax.bitcast_convert_type(x, jnp.int32)
    i = jnp.int32(0x5F3759DF) - lax.shift_right_logical(i, 1)
    y = lax.bitcast_convert_type(i, jnp.float32)
    for _ in range(2):
        y = y * (1.5 - 0.5 * x * y * y)
    return y


def _body(tok_hbm, tab_hbm, pos_hbm, lns_hbm, lnb_hbm, tail_hbm, out_hbm,
          idx_v, hidx_v, rows_v, res_v, pos_v, lns_v, lnb_v, tail_v,
          gsem0, gsem1, osem0, osem1):
    wid = lax.axis_index("s") * _NC + lax.axis_index("c")
    pltpu.sync_copy(pos_hbm, pos_v)
    pltpu.sync_copy(lns_hbm, lns_v)
    pltpu.sync_copy(lnb_hbm, lnb_v)
    pltpu.sync_copy(tail_hbm, tail_v)
    lane = lax.iota(jnp.int32, 16)
    sc_regs = [lns_v[pl.ds(k * 16, 16)] for k in range(4)]
    bi_regs = [lnb_v[pl.ds(k * 16, 16)] for k in range(4)]
    base0 = wid * _PER_W

    def gather_copies(boff, gsem):
        return [
            pltpu.make_async_copy(
                tab_hbm.at[hidx_v.at[pl.ds(boff + i * _GS, _GS)]],
                rows_v.at[pl.ds(boff + i * _GS, _GS)], gsem)
            for i in range(_NG)
        ]

    def fire_gather(c, boff, gsem):
        pltpu.sync_copy(tok_hbm.at[pl.ds(base0 + c * _CH, _CH)],
                        idx_v.at[pl.ds(boff, _CH)])
        for i in range(_CH // 16):
            t = idx_v[pl.ds(boff + i * 16, 16)]
            h = jnp.where(t < jnp.int32(_P),
                          t * jnp.int32(2),
                          (t - jnp.int32(_P)) * jnp.int32(2) + jnp.int32(1))
            hidx_v[pl.ds(boff + i * 16, 16)] = jnp.minimum(
                h, jnp.int32(2 * _PROWS - 1))
        for cp in gather_copies(boff, gsem):
            cp.start()

    def drain_gather(boff, gsem):
        for cp in gather_copies(boff, gsem):
            cp.wait()

    ivecs = [jnp.int32(2 * k) + lax.shift_right_logical(lane, 3)
             for k in range(4)]
    divec = lax.bitwise_and(lane, jnp.int32(7))

    def out_copy(c, half, osem):
        return pltpu.make_async_copy(
            res_v.at[half],
            out_hbm.at[:, :, wid, :, pl.ds(c * _SEQ_CH, _SEQ_CH)], osem)

    def compute(c, boff, half):
        hvec = jnp.broadcast_to(half, (16,)).astype(jnp.int32)

        def token_body(t0, j, xs, lvec, svec, tokv=None):
            if tokv is not None:
                ti = jnp.clip(tokv[j] - jnp.int32(_TAIL0),
                              0, _V - _TAIL0 - 1) * jnp.int32(_D)
                tm = jnp.broadcast_to(tokv[j] >= jnp.int32(_TAIL0), (16,))
                for k in range(4):
                    tk = tail_v[pl.ds(ti + k * 16, 16)]
                    xs[k] = jnp.where(tm, tk, xs[k])
            l_j = jnp.broadcast_to(lvec[j], (16,))
            s_j = jnp.broadcast_to(svec[j], (16,))
            ssum = _allreduce_sum((xs[0] + xs[1]) + (xs[2] + xs[3]), lane)
            qsum = _allreduce_sum(
                (xs[0] * xs[0] + xs[1] * xs[1])
                + (xs[2] * xs[2] + xs[3] * xs[3]), lane)
            mean = ssum * jnp.float32(1.0 / _D)
            var = qsum * jnp.float32(1.0 / _D) - mean * mean
            inv = _rsqrt(var + jnp.float32(_EPS))
            for k in range(4):
                yk = (xs[k] - mean) * inv * sc_regs[k] + bi_regs[k]
                plsc.store_scatter(
                    res_v, [hvec, l_j, ivecs[k], divec, s_j], yk)

        def scan_pad(g, acc):
            tokv = idx_v[pl.ds(boff + g * 16, 16)]
            bad = jnp.logical_or(tokv == jnp.int32(0),
                                 tokv >= jnp.int32(_TAIL0))
            return jnp.logical_or(acc, jnp.any(bad))

        has_pad = lax.fori_loop(0, _CH // 16, scan_pad, False, unroll=False)

        @pl.when(jnp.logical_not(has_pad))
        def _():
            @plsc.parallel_loop(0, _CH // 16, unroll=2)
            def group(g):
                t0 = boff + g * 16
                gt = g * 16 + lane
                lvec = lax.rem(gt, jnp.int32(_L))
                svec = lax.div(gt, jnp.int32(_L))
                for j in range(16):
                    xs = [rows_v[t0 + j, pl.ds(k * 16, 16)]
                          for k in range(4)]
                    token_body(t0, j, xs, lvec, svec)

        @pl.when(has_pad)
        def _():
            @plsc.parallel_loop(0, _CH // 16, unroll=1)
            def group(g):
                t0 = boff + g * 16
                tokv = idx_v[pl.ds(t0, 16)]
                gt = g * 16 + lane
                lvec = lax.rem(gt, jnp.int32(_L))
                svec = lax.div(gt, jnp.int32(_L))
                posv = jnp.where(tokv != jnp.int32(0), jnp.int32(0), lvec)
                pflat = posv * jnp.int32(_D)
                for j in range(16):
                    p_j = pflat[j]
                    xs = []
                    for k in range(4):
                        xk = rows_v[t0 + j, pl.ds(k * 16, 16)]
                        pk = pos_v[pl.ds(p_j + k * 16, 16)]
                        xs.append(xk + pk)
                    token_body(t0, j, xs, lvec, svec, tokv)

    fire_gather(0, 0, gsem0)

    def chunk_step(cc, carry):
        even = lax.rem(cc, 2) == 0
        boff = lax.rem(cc, 2) * _CH

        @pl.when(even)
        def _():
            drain_gather(0, gsem0)

        @pl.when(jnp.logical_not(even))
        def _():
            drain_gather(_CH, gsem1)

        @pl.when(jnp.logical_and(even, cc + 1 < _NCH))
        def _():
            fire_gather(cc + 1, _CH, gsem1)

        @pl.when(jnp.logical_and(jnp.logical_not(even), cc + 1 < _NCH))
        def _():
            fire_gather(cc + 1, 0, gsem0)

        @pl.when(jnp.logical_and(even, cc >= 2))
        def _():
            out_copy(cc - 2, 0, osem0).wait()

        @pl.when(jnp.logical_and(jnp.logical_not(even), cc >= 2))
        def _():
            out_copy(cc - 2, 1, osem1).wait()

        compute(cc, boff, lax.rem(cc, 2))

        @pl.when(even)
        def _():
            out_copy(cc, 0, osem0).start()

        @pl.when(jnp.logical_not(even))
        def _():
            out_copy(cc, 1, osem1).start()

        return carry

    lax.fori_loop(0, _NCH, chunk_step, 0, unroll=False)
    out_copy(_NCH - 2, 0, osem0).wait()
    out_copy(_NCH - 1, 1, osem1).wait()


def kernel(tokens, token_table, pos_table, ln_scale, ln_bias):
    tokens_flat = tokens.astype(jnp.int32).reshape(_N)
    pos_flat = pos_table.reshape(_L * _D)
    mesh = plsc.VectorSubcoreMesh(core_axis_name="c", subcore_axis_name="s")
    k = pl.kernel(
        _body,
        out_type=jax.ShapeDtypeStruct((_L, 8, _NW, 8, 128), jnp.float32),
        mesh=mesh,
        compiler_params=pltpu.CompilerParams(
            needs_layout_passes=False, use_tc_tiling_on_sc=False),
        scratch_types=[
            pltpu.VMEM((2 * _CH,), jnp.int32),
            pltpu.VMEM((2 * _CH,), jnp.int32),
            pltpu.VMEM((2 * _CH, _D), jnp.float32),
            pltpu.VMEM((2, _L, 8, 8, _SEQ_CH), jnp.float32),
            pltpu.VMEM((_L * _D,), jnp.float32),
            pltpu.VMEM((_D,), jnp.float32),
            pltpu.VMEM((_D,), jnp.float32),
            pltpu.VMEM(((_V - _TAIL0) * _D,), jnp.float32),
            pltpu.SemaphoreType.DMA,
            pltpu.SemaphoreType.DMA,
            pltpu.SemaphoreType.DMA,
            pltpu.SemaphoreType.DMA,
        ],
    )
    pairs = _pack_pairs(token_table.T).reshape(2 * _PROWS, _D)
    tail = token_table[_TAIL0:].reshape((_V - _TAIL0) * _D)
    out5 = k(tokens_flat, pairs, pos_flat, ln_scale, ln_bias, tail)
    return out5.transpose(2, 4, 0, 1, 3).reshape(_B, _L, _D)


def _tc_pack_kernel(tabt_ref, a_ref, o_ref, b_v, sem):
    i = pl.program_id(0)
    cp = pltpu.make_async_copy(
        tabt_ref.at[:, pl.ds(_P + i * _TBLK, _TBLK)], b_v, sem)
    cp.start()
    o_ref[:, pl.ds(0, _D)] = a_ref[...].T
    cp.wait()
    o_ref[:, pl.ds(_D, _D)] = b_v[...].T


def _pack_pairs(table_t):
    return pl.pallas_call(
        _tc_pack_kernel,
        grid=(_PROWS // _TBLK,),
        in_specs=[
            pl.BlockSpec(memory_space=pl.ANY),
            pl.BlockSpec((_D, _TBLK), lambda i: (0, i)),
        ],
        out_specs=pl.BlockSpec((_TBLK, 2 * _D), lambda i: (i, 0)),
        out_shape=jax.ShapeDtypeStruct((_PROWS, 2 * _D), jnp.float32),
        scratch_shapes=[
            pltpu.VMEM((_D, _TBLK), jnp.float32),
            pltpu.SemaphoreType.DMA,
        ],
    )(table_t, table_t)
